# baseline (speedup 1.0000x reference)
import jax
import jax.numpy as jnp
from jax import lax
from jax.experimental import pallas as pl
from jax.experimental.pallas import tpu as pltpu
from jax.experimental.pallas import tpu_sc as plsc

_NUM_EMB = 1000000
_EMB_DIM = 16
_BATCH = 16384
_HIST = 50

_NC = 2
_NS = 16
_NW = _NC * _NS
_CU = 4
_BLK = _CU * 128
_PLANE = _CU * 1024


def _b_body(wt_hbm, table_hbm, out3,
            ibuf0, ibuf1, rbuf0, rbuf1, tbuf0, tbuf1,
            si, sg0, sg1, sw0, sw1):
    wid = lax.axis_index("s") * _NC + lax.axis_index("c")
    cu0 = wid * _CU
    b0 = wid * _BLK

    ibufs = (ibuf0, ibuf1)
    rbufs = (rbuf0, rbuf1)
    tbufs = (tbuf0, tbuf1)
    sgs = (sg0, sg1)
    sws = (sw0, sw1)

    lane = lax.iota(jnp.int32, 16)
    static_vec = (lane // 8) * _PLANE + (lane % 8) * 128

    def fire_idx(h, par):
        pltpu.async_copy(wt_hbm.at[h, pl.ds(b0, _BLK)], ibufs[par], si)

    def wait_idx(par):
        pltpu.make_async_copy(wt_hbm.at[0, pl.ds(b0, _BLK)], ibufs[par], si
                              ).wait()

    def fire_gathers(par):
        for q in range(_CU):
            pltpu.async_copy(
                table_hbm.at[ibufs[par].at[pl.ds(128 * q, 128)]],
                rbufs[par].at[q], sgs[par])

    def wait_gathers(par):
        for q in range(_CU):
            pltpu.make_async_copy(
                table_hbm.at[ibufs[par].at[pl.ds(128 * q, 128)]],
                rbufs[par].at[q], sgs[par]).wait()

    def fire_writes(h, par):
        for su in range(2):
            pltpu.async_copy(
                tbufs[par].at[pl.ds(su * _PLANE, _PLANE)],
                out3.at[h, su, pl.ds(cu0 * 1024, _PLANE)], sws[par])

    def wait_writes(h, par):
        for su in range(2):
            pltpu.make_async_copy(
                tbufs[par].at[pl.ds(su * _PLANE, _PLANE)],
                out3.at[h, su, pl.ds(cu0 * 1024, _PLANE)], sws[par]).wait()

    def transpose(par):
        rbuf = rbufs[par]
        tbuf = tbufs[par]

        dvs = []
        for j in range(_EMB_DIM):
            t = j + lane
            dvj = jnp.where(t >= _EMB_DIM, t - _EMB_DIM, t)
            posd = ((dvj >> 3) * _PLANE) + ((dvj & 7) * 128)
            dvs.append((dvj, posd))

        def trans(lb, _):
            lvec = lb * 16 + lane
            for q in range(_CU):
                qv = jnp.full((16,), q, jnp.int32)
                base = (q * 1024) + lvec
                for dvj, posd in dvs:
                    v = plsc.load_gather(rbuf, [qv, lvec, dvj])
                    plsc.store_scatter(tbuf, [posd + base], v)
            return 0

        lax.fori_loop(0, 8, trans, 0)

    pltpu.sync_copy(wt_hbm.at[0, pl.ds(b0, _BLK)], ibufs[0])
    fire_gathers(0)
    fire_idx(1, 1)

    def step(hh, _):
        for par in range(2):
            h = 2 * hh + par
            nxt = 1 - par

            @pl.when(h + 1 < _HIST)
            def _():
                wait_idx(nxt)

                @pl.when(h >= 1)
                def _():
                    wait_writes(h, nxt)

                fire_gathers(nxt)

            wait_gathers(par)

            @pl.when(h + 2 < _HIST)
            def _():
                fire_idx(h + 2, par)

            transpose(par)
            fire_writes(h, par)
        return 0

    lax.fori_loop(0, _HIST // 2, step, 0)
    wait_writes(_HIST - 2, 0)
    wait_writes(_HIST - 1, 1)


_gather = pl.kernel(
    _b_body,
    out_type=jax.ShapeDtypeStruct((_HIST, 2, 128 * 1024), jnp.float32),
    mesh=plsc.VectorSubcoreMesh(core_axis_name="c", subcore_axis_name="s"),
    scratch_types=[
        pltpu.VMEM((_BLK,), jnp.int32),
        pltpu.VMEM((_BLK,), jnp.int32),
        pltpu.VMEM((_CU, 128, _EMB_DIM), jnp.float32),
        pltpu.VMEM((_CU, 128, _EMB_DIM), jnp.float32),
        pltpu.VMEM((2 * _PLANE,), jnp.float32),
        pltpu.VMEM((2 * _PLANE,), jnp.float32),
        pltpu.SemaphoreType.DMA,
        pltpu.SemaphoreType.DMA,
        pltpu.SemaphoreType.DMA,
        pltpu.SemaphoreType.DMA,
        pltpu.SemaphoreType.DMA,
    ],
    compiler_params=pltpu.CompilerParams(
        use_tc_tiling_on_sc=False, needs_layout_passes=False),
)


def kernel(weights, emb_table):
    wT = weights.T
    out3 = _gather(wT, emb_table)
    out5 = out3.reshape(_HIST, 2, 128, 8, 128)
    return out5.transpose(2, 4, 0, 1, 3).reshape(_BATCH, _HIST, _EMB_DIM)


# speedup vs baseline: 1.0657x; 1.0657x over previous
import jax
import jax.numpy as jnp
from jax import lax
from jax.experimental import pallas as pl
from jax.experimental.pallas import tpu as pltpu
from jax.experimental.pallas import tpu_sc as plsc

_NUM_EMB = 1000000
_EMB_DIM = 16
_BATCH = 16384
_HIST = 50

_NC = 2
_NS = 16
_NW = _NC * _NS
_CU = 4
_BLK = _CU * 128
_PLANE = _CU * 1024


def _b_body(wt_hbm, table_hbm, out3,
            ibuf0, ibuf1, rbuf0, rbuf1, tbuf0, tbuf1,
            si, sg0, sg1, sw0, sw1):
    wid = lax.axis_index("s") * _NC + lax.axis_index("c")
    cu0 = wid * _CU
    b0 = wid * _BLK

    ibufs = (ibuf0, ibuf1)
    rbufs = (rbuf0, rbuf1)
    tbufs = (tbuf0, tbuf1)
    sgs = (sg0, sg1)
    sws = (sw0, sw1)

    lane = lax.iota(jnp.int32, 16)
    static_vec = (lane // 8) * _PLANE + (lane % 8) * 128

    def fire_idx(h, par):
        pltpu.async_copy(wt_hbm.at[h, pl.ds(b0, _BLK)], ibufs[par], si)

    def wait_idx(par):
        pltpu.make_async_copy(wt_hbm.at[0, pl.ds(b0, _BLK)], ibufs[par], si
                              ).wait()

    def fire_gathers(par):
        pltpu.async_copy(
            table_hbm.at[ibufs[par]], rbufs[par], sgs[par])

    def wait_gathers(par):
        pltpu.make_async_copy(
            table_hbm.at[ibufs[par]], rbufs[par], sgs[par]).wait()

    def fire_writes(h, par):
        for su in range(2):
            pltpu.async_copy(
                tbufs[par].at[pl.ds(su * _PLANE, _PLANE)],
                out3.at[h, su, pl.ds(cu0 * 1024, _PLANE)], sws[par])

    def wait_writes(h, par):
        for su in range(2):
            pltpu.make_async_copy(
                tbufs[par].at[pl.ds(su * _PLANE, _PLANE)],
                out3.at[h, su, pl.ds(cu0 * 1024, _PLANE)], sws[par]).wait()

    def transpose(par):
        rbuf = rbufs[par]
        tbuf = tbufs[par]

        for j in range(_EMB_DIM):
            t = j + lane
            dvj = jnp.where(t >= _EMB_DIM, t - _EMB_DIM, t)
            posd = ((dvj >> 3) * _PLANE) + ((dvj & 7) * 128)

            def trans_j(lb, _, dvj=dvj, posd=posd):
                lvec = lb * 16 + lane
                for q in range(_CU):
                    v = plsc.load_gather(rbuf, [q * 128 + lvec, dvj])
                    sidx = posd + (q * 1024) + lvec
                    plsc.store_scatter(tbuf, [sidx], v)
                return 0

            lax.fori_loop(0, 8, trans_j, 0)

    pltpu.sync_copy(wt_hbm.at[0, pl.ds(b0, _BLK)], ibufs[0])
    fire_gathers(0)
    fire_idx(1, 1)

    def step(hh, _):
        for par in range(2):
            h = 2 * hh + par
            nxt = 1 - par

            @pl.when(h + 1 < _HIST)
            def _():
                wait_idx(nxt)

                @pl.when(h >= 1)
                def _():
                    wait_writes(h, nxt)

                fire_gathers(nxt)

            wait_gathers(par)

            @pl.when(h + 2 < _HIST)
            def _():
                fire_idx(h + 2, par)

            transpose(par)
            fire_writes(h, par)
        return 0

    lax.fori_loop(0, _HIST // 2, step, 0)
    wait_writes(_HIST - 2, 0)
    wait_writes(_HIST - 1, 1)


_gather = pl.kernel(
    _b_body,
    out_type=jax.ShapeDtypeStruct((_HIST, 2, 128 * 1024), jnp.float32),
    mesh=plsc.VectorSubcoreMesh(core_axis_name="c", subcore_axis_name="s"),
    scratch_types=[
        pltpu.VMEM((_BLK,), jnp.int32),
        pltpu.VMEM((_BLK,), jnp.int32),
        pltpu.VMEM((_BLK, _EMB_DIM), jnp.float32),
        pltpu.VMEM((_BLK, _EMB_DIM), jnp.float32),
        pltpu.VMEM((2 * _PLANE,), jnp.float32),
        pltpu.VMEM((2 * _PLANE,), jnp.float32),
        pltpu.SemaphoreType.DMA,
        pltpu.SemaphoreType.DMA,
        pltpu.SemaphoreType.DMA,
        pltpu.SemaphoreType.DMA,
        pltpu.SemaphoreType.DMA,
    ],
    compiler_params=pltpu.CompilerParams(
        use_tc_tiling_on_sc=False, needs_layout_passes=False),
)


def kernel(weights, emb_table):
    wT = weights.T
    out3 = _gather(wT, emb_table)
    out5 = out3.reshape(_HIST, 2, 128, 8, 128)
    return out5.transpose(2, 4, 0, 1, 3).reshape(_BATCH, _HIST, _EMB_DIM)


# speedup vs baseline: 1.0667x; 1.0009x over previous
import jax
import jax.numpy as jnp
from jax import lax
from jax.experimental import pallas as pl
from jax.experimental.pallas import tpu as pltpu
from jax.experimental.pallas import tpu_sc as plsc

_NUM_EMB = 1000000
_EMB_DIM = 16
_BATCH = 16384
_HIST = 50

_NC = 2
_NS = 16
_NW = _NC * _NS
_CU = 4
_BLK = _CU * 128
_PLANE = _CU * 1024


def _b_body(wt_hbm, table_hbm, out3,
            ibuf0, ibuf1, rbuf0, rbuf1, tbuf0, tbuf1,
            si, sg0, sg1, sw0, sw1):
    wid = lax.axis_index("s") * _NC + lax.axis_index("c")
    cu0 = wid * _CU
    b0 = wid * _BLK

    ibufs = (ibuf0, ibuf1)
    rbufs = (rbuf0, rbuf1)
    tbufs = (tbuf0, tbuf1)
    sgs = (sg0, sg1)
    sws = (sw0, sw1)

    lane = lax.iota(jnp.int32, 16)

    def fire_idx(h, par):
        pltpu.async_copy(wt_hbm.at[h, pl.ds(b0, _BLK)], ibufs[par], si)

    def wait_idx(par):
        pltpu.make_async_copy(wt_hbm.at[0, pl.ds(b0, _BLK)], ibufs[par], si
                              ).wait()

    def fire_gathers(par):
        pltpu.async_copy(
            table_hbm.at[ibufs[par]], rbufs[par], sgs[par])

    def wait_gathers(par):
        pltpu.make_async_copy(
            table_hbm.at[ibufs[par]], rbufs[par], sgs[par]).wait()

    def fire_writes(h, par):
        for su in range(2):
            pltpu.async_copy(
                tbufs[par].at[pl.ds(su * _PLANE, _PLANE)],
                out3.at[h, su, pl.ds(cu0 * 1024, _PLANE)], sws[par])

    def wait_writes(h, par):
        for su in range(2):
            pltpu.make_async_copy(
                tbufs[par].at[pl.ds(su * _PLANE, _PLANE)],
                out3.at[h, su, pl.ds(cu0 * 1024, _PLANE)], sws[par]).wait()

    def transpose(par):
        rbuf = rbufs[par]
        tbuf = tbufs[par]

        for j in range(_EMB_DIM):
            t = j + lane
            dvj = jnp.where(t >= _EMB_DIM, t - _EMB_DIM, t)
            posd = ((dvj >> 3) * _PLANE) + ((dvj & 7) * 128)

            def trans_j(lb, _, dvj=dvj, posd=posd):
                lvec = lb * 16 + lane
                for q in range(_CU):
                    v = plsc.load_gather(rbuf, [q * 128 + lvec, dvj])
                    sidx = posd + (q * 1024) + lvec
                    plsc.store_scatter(tbuf, [sidx], v)
                return 0

            lax.fori_loop(0, 8, trans_j, 0)

    pltpu.sync_copy(wt_hbm.at[0, pl.ds(b0, _BLK)], ibufs[0])
    fire_gathers(0)
    fire_idx(1, 1)

    def step(hh, _):
        for par in range(2):
            h = 2 * hh + par
            nxt = 1 - par

            @pl.when(h + 1 < _HIST)
            def _():
                wait_idx(nxt)

                @pl.when(h >= 1)
                def _():
                    wait_writes(h, nxt)

                fire_gathers(nxt)

            wait_gathers(par)

            @pl.when(h + 2 < _HIST)
            def _():
                fire_idx(h + 2, par)

            transpose(par)
            fire_writes(h, par)
        return 0

    lax.fori_loop(0, _HIST // 2, step, 0)
    wait_writes(_HIST - 2, 0)
    wait_writes(_HIST - 1, 1)


_gather = pl.kernel(
    _b_body,
    out_type=jax.ShapeDtypeStruct((_HIST, 2, 128 * 1024), jnp.float32),
    mesh=plsc.VectorSubcoreMesh(core_axis_name="c", subcore_axis_name="s"),
    scratch_types=[
        pltpu.VMEM((_BLK,), jnp.int32),
        pltpu.VMEM((_BLK,), jnp.int32),
        pltpu.VMEM((_BLK, _EMB_DIM), jnp.float32),
        pltpu.VMEM((_BLK, _EMB_DIM), jnp.float32),
        pltpu.VMEM((2 * _PLANE,), jnp.float32),
        pltpu.VMEM((2 * _PLANE,), jnp.float32),
        pltpu.SemaphoreType.DMA,
        pltpu.SemaphoreType.DMA,
        pltpu.SemaphoreType.DMA,
        pltpu.SemaphoreType.DMA,
        pltpu.SemaphoreType.DMA,
    ],
    compiler_params=pltpu.CompilerParams(
        use_tc_tiling_on_sc=False, needs_layout_passes=False),
)


def kernel(weights, emb_table):
    wT = weights.T
    out3 = _gather(wT, emb_table)
    out5 = out3.reshape(_HIST, 2, 128, 8, 128)
    return out5.transpose(2, 4, 0, 1, 3).reshape(_BATCH, _HIST, _EMB_DIM)
